# baseline (speedup 1.0000x reference)
import functools

import jax
import jax.numpy as jnp
from jax import lax
from jax.experimental import pallas as pl
from jax.experimental.pallas import tpu as pltpu
from jax.experimental.pallas import tpu_sc as plsc

T = 128
HIDDEN = 1024
E = 64
TOPK = 2
LANES = 16
NCHUNK = E // LANES


def _gate_body(x_ref, Wg_ref, bg_ref, gate_ref):
    logits = jnp.dot(x_ref[...], Wg_ref[...], preferred_element_type=jnp.float32)
    logits = logits + bg_ref[...]
    m = jnp.max(logits, axis=1, keepdims=True)
    p = jnp.exp(logits - m)
    gate_ref[...] = p / jnp.sum(p, axis=1, keepdims=True)


def _top2_sc_body(gate_hbm, wmat_hbm, rows_v, out_v):
    info = plsc.get_sparse_core_info()
    nc = info.num_cores
    wid = lax.axis_index("s") * nc + lax.axis_index("c")
    tpw = T // (nc * info.num_subcores)
    base = wid * tpw
    pltpu.sync_copy(gate_hbm.at[pl.ds(base, tpw)], rows_v)
    iota = lax.iota(jnp.int32, LANES)
    for t in range(tpw):
        chunks = [rows_v[t, pl.ds(c * LANES, LANES)] for c in range(NCHUNK)]
        m1 = jnp.float32(-1.0)
        a1 = jnp.int32(0)
        m2 = jnp.float32(-1.0)
        a2 = jnp.int32(0)
        for i in range(E):
            g = chunks[i // LANES][i % LANES]
            take1 = g > m1
            take2 = jnp.logical_and(jnp.logical_not(take1), g > m2)
            a2 = jnp.where(take1, a1, jnp.where(take2, i, a2))
            m2 = jnp.where(take1, m1, jnp.where(take2, g, m2))
            a1 = jnp.where(take1, i, a1)
            m1 = jnp.where(take1, g, m1)
        for c in range(NCHUNK):
            gidx = iota + c * LANES
            w = (jnp.where(gidx == a1, m1, 0.0)
                 + jnp.where(gidx == a2, m2, 0.0))
            out_v[t, pl.ds(c * LANES, LANES)] = w
    pltpu.sync_copy(out_v, wmat_hbm.at[pl.ds(base, tpw)])


def _expert_body(x_ref, wmat_ref, W1_ref, b1_ref, W2_ref, b2_ref, out_ref):
    e = pl.program_id(0)

    @pl.when(e == 0)
    def _init():
        out_ref[...] = jnp.zeros_like(out_ref)

    onehot = (lax.broadcasted_iota(jnp.int32, (E, 1), 0) == e).astype(jnp.float32)
    col = jnp.dot(wmat_ref[...], onehot, preferred_element_type=jnp.float32)

    @pl.when(jnp.sum(col) > 0.0)
    def _expert():
        xb = x_ref[...].astype(jnp.bfloat16)
        h = jnp.dot(xb, W1_ref[0].astype(jnp.bfloat16),
                    preferred_element_type=jnp.float32)
        h = jnp.maximum(h + b1_ref[0], 0.0)
        y = jnp.dot(h.astype(jnp.bfloat16), W2_ref[0].astype(jnp.bfloat16),
                    preferred_element_type=jnp.float32)
        y = y + b2_ref[0]
        out_ref[...] += col * y


def kernel(x, Wg, bg, W1, b1, W2, b2):
    bg2 = bg.reshape(1, E)
    b1 = b1.reshape(E, 1, HIDDEN)
    b2 = b2.reshape(E, 1, HIDDEN)

    gate = pl.pallas_call(
        _gate_body,
        in_specs=[
            pl.BlockSpec((T, HIDDEN), lambda: (0, 0)),
            pl.BlockSpec((HIDDEN, E), lambda: (0, 0)),
            pl.BlockSpec((1, E), lambda: (0, 0)),
        ],
        out_specs=pl.BlockSpec((T, E), lambda: (0, 0)),
        out_shape=jax.ShapeDtypeStruct((T, E), jnp.float32),
    )(x, Wg, bg2)

    mesh = plsc.VectorSubcoreMesh(core_axis_name="c", subcore_axis_name="s")
    info = plsc.get_sparse_core_info()
    tpw = T // (info.num_cores * info.num_subcores)
    top2 = functools.partial(
        pl.kernel,
        mesh=mesh,
        out_type=jax.ShapeDtypeStruct((T, E), jnp.float32),
        scratch_types=[
            pltpu.VMEM((tpw, E), jnp.float32),
            pltpu.VMEM((tpw, E), jnp.float32),
        ],
    )(_top2_sc_body)
    wmat = top2(gate)

    out = pl.pallas_call(
        _expert_body,
        grid=(E,),
        in_specs=[
            pl.BlockSpec((T, HIDDEN), lambda e: (0, 0)),
            pl.BlockSpec((T, E), lambda e: (0, 0)),
            pl.BlockSpec((1, HIDDEN, HIDDEN), lambda e: (e, 0, 0)),
            pl.BlockSpec((1, 1, HIDDEN), lambda e: (e, 0, 0)),
            pl.BlockSpec((1, HIDDEN, HIDDEN), lambda e: (e, 0, 0)),
            pl.BlockSpec((1, 1, HIDDEN), lambda e: (e, 0, 0)),
        ],
        out_specs=pl.BlockSpec((T, HIDDEN), lambda e: (0, 0)),
        out_shape=jax.ShapeDtypeStruct((T, HIDDEN), jnp.float32),
    )(x, wmat, W1, b1, W2, b2)
    return (out, gate)


# speedup vs baseline: 1.1048x; 1.1048x over previous
import jax
import jax.numpy as jnp
from jax import lax
from jax.experimental import pallas as pl
from jax.experimental.pallas import tpu as pltpu

T = 128
HIDDEN = 1024
E = 64
TOPK = 2


def _moe_body(x_ref, Wg_ref, bg_ref, W1_ref, b1_ref, W2_ref, b2_ref,
              out_ref, gate_ref, wmat_ref):
    e = pl.program_id(0)

    @pl.when(e == 0)
    def _gate():
        x = x_ref[...]
        logits = jnp.dot(x, Wg_ref[...], preferred_element_type=jnp.float32)
        logits = logits + bg_ref[...]
        m = jnp.max(logits, axis=1, keepdims=True)
        p = jnp.exp(logits - m)
        gate = p / jnp.sum(p, axis=1, keepdims=True)
        gate_ref[...] = gate

        iota_e = lax.broadcasted_iota(jnp.int32, (T, E), 1)
        m1 = jnp.max(gate, axis=1, keepdims=True)
        a1 = jnp.min(jnp.where(gate == m1, iota_e, E), axis=1, keepdims=True)
        sel1 = iota_e == a1
        gate2 = jnp.where(sel1, -1.0, gate)
        m2 = jnp.max(gate2, axis=1, keepdims=True)
        a2 = jnp.min(jnp.where(gate2 == m2, iota_e, E), axis=1, keepdims=True)
        sel2 = iota_e == a2
        wmat_ref[...] = jnp.where(sel1, m1, 0.0) + jnp.where(sel2, m2, 0.0)
        out_ref[...] = jnp.zeros_like(out_ref)

    onehot = (lax.broadcasted_iota(jnp.int32, (E, 1), 0) == e).astype(jnp.float32)
    col = jnp.dot(wmat_ref[...], onehot, preferred_element_type=jnp.float32)

    @pl.when(jnp.sum(col) > 0.0)
    def _expert():
        h = jnp.dot(x_ref[...], W1_ref[0], preferred_element_type=jnp.float32)
        h = jnp.maximum(h + b1_ref[0], 0.0)
        y = jnp.dot(h, W2_ref[0], preferred_element_type=jnp.float32)
        y = y + b2_ref[0]
        out_ref[...] += col * y


def kernel(x, Wg, bg, W1, b1, W2, b2):
    bg2 = bg.reshape(1, E)
    b1 = b1.reshape(E, 1, HIDDEN)
    b2 = b2.reshape(E, 1, HIDDEN)
    out, gate = pl.pallas_call(
        _moe_body,
        grid=(E,),
        in_specs=[
            pl.BlockSpec((T, HIDDEN), lambda e: (0, 0)),
            pl.BlockSpec((HIDDEN, E), lambda e: (0, 0)),
            pl.BlockSpec((1, E), lambda e: (0, 0)),
            pl.BlockSpec((1, HIDDEN, HIDDEN), lambda e: (e, 0, 0)),
            pl.BlockSpec((1, 1, HIDDEN), lambda e: (e, 0, 0)),
            pl.BlockSpec((1, HIDDEN, HIDDEN), lambda e: (e, 0, 0)),
            pl.BlockSpec((1, 1, HIDDEN), lambda e: (e, 0, 0)),
        ],
        out_specs=[
            pl.BlockSpec((T, HIDDEN), lambda e: (0, 0)),
            pl.BlockSpec((T, E), lambda e: (0, 0)),
        ],
        out_shape=[
            jax.ShapeDtypeStruct((T, HIDDEN), jnp.float32),
            jax.ShapeDtypeStruct((T, E), jnp.float32),
        ],
        scratch_shapes=[pltpu.VMEM((T, E), jnp.float32)],
    )(x, Wg, bg2, W1, b1, W2, b2)
    return (out, gate)


# speedup vs baseline: 1.1177x; 1.0116x over previous
import jax
import jax.numpy as jnp
from jax import lax
from jax.experimental import pallas as pl
from jax.experimental.pallas import tpu as pltpu

T = 128
HIDDEN = 1024
E = 64
TOPK = 2


def _moe_body(x_ref, Wg_ref, bg_ref, W1_ref, b1_ref, W2_ref, b2_ref,
              out_ref, gate_ref, wmat_ref):
    e = pl.program_id(0)

    @pl.when(e == 0)
    def _gate():
        x = x_ref[...]
        logits = jnp.dot(x, Wg_ref[...], preferred_element_type=jnp.float32)
        logits = logits + bg_ref[...]
        m = jnp.max(logits, axis=1, keepdims=True)
        p = jnp.exp(logits - m)
        gate = p / jnp.sum(p, axis=1, keepdims=True)
        gate_ref[...] = gate

        iota_e = lax.broadcasted_iota(jnp.int32, (T, E), 1)
        m1 = jnp.max(gate, axis=1, keepdims=True)
        a1 = jnp.min(jnp.where(gate == m1, iota_e, E), axis=1, keepdims=True)
        sel1 = iota_e == a1
        gate2 = jnp.where(sel1, -1.0, gate)
        m2 = jnp.max(gate2, axis=1, keepdims=True)
        a2 = jnp.min(jnp.where(gate2 == m2, iota_e, E), axis=1, keepdims=True)
        sel2 = iota_e == a2
        wmat_ref[...] = jnp.where(sel1, m1, 0.0) + jnp.where(sel2, m2, 0.0)
        out_ref[...] = jnp.zeros_like(out_ref)

    onehot = (lax.broadcasted_iota(jnp.int32, (E, 1), 0) == e).astype(jnp.float32)
    col = jnp.dot(wmat_ref[...], onehot, preferred_element_type=jnp.float32)

    h = jnp.dot(x_ref[...], W1_ref[0], preferred_element_type=jnp.float32)
    h = jnp.maximum(h + b1_ref[0], 0.0)
    y = jnp.dot(h, W2_ref[0], preferred_element_type=jnp.float32)
    y = y + b2_ref[0]
    out_ref[...] += col * y


def kernel(x, Wg, bg, W1, b1, W2, b2):
    bg2 = bg.reshape(1, E)
    b1 = b1.reshape(E, 1, HIDDEN)
    b2 = b2.reshape(E, 1, HIDDEN)
    out, gate = pl.pallas_call(
        _moe_body,
        grid=(E,),
        in_specs=[
            pl.BlockSpec((T, HIDDEN), lambda e: (0, 0)),
            pl.BlockSpec((HIDDEN, E), lambda e: (0, 0)),
            pl.BlockSpec((1, E), lambda e: (0, 0)),
            pl.BlockSpec((1, HIDDEN, HIDDEN), lambda e: (e, 0, 0)),
            pl.BlockSpec((1, 1, HIDDEN), lambda e: (e, 0, 0)),
            pl.BlockSpec((1, HIDDEN, HIDDEN), lambda e: (e, 0, 0)),
            pl.BlockSpec((1, 1, HIDDEN), lambda e: (e, 0, 0)),
        ],
        out_specs=[
            pl.BlockSpec((T, HIDDEN), lambda e: (0, 0)),
            pl.BlockSpec((T, E), lambda e: (0, 0)),
        ],
        out_shape=[
            jax.ShapeDtypeStruct((T, HIDDEN), jnp.float32),
            jax.ShapeDtypeStruct((T, E), jnp.float32),
        ],
        scratch_shapes=[pltpu.VMEM((T, E), jnp.float32)],
    )(x, Wg, bg2, W1, b1, W2, b2)
    return (out, gate)


# speedup vs baseline: 1.1441x; 1.0236x over previous
import jax
import jax.numpy as jnp
from jax import lax
from jax.experimental import pallas as pl
from jax.experimental.pallas import tpu as pltpu

T = 128
HIDDEN = 1024
E = 64
TOPK = 2
NBUF = 3


def _moe_body(x_ref, Wg_ref, bg_ref, b1_ref, b2_ref, W1_hbm, W2_hbm,
              out_ref, gate_ref, w1buf, w2buf, wmat_ref, sem):
    x = x_ref[...]
    logits = jnp.dot(x, Wg_ref[...], preferred_element_type=jnp.float32)
    logits = logits + bg_ref[...]
    m = jnp.max(logits, axis=1, keepdims=True)
    p = jnp.exp(logits - m)
    gate = p / jnp.sum(p, axis=1, keepdims=True)
    gate_ref[...] = gate

    iota_e = lax.broadcasted_iota(jnp.int32, (T, E), 1)
    m1 = jnp.max(gate, axis=1, keepdims=True)
    a1 = jnp.min(jnp.where(gate == m1, iota_e, E), axis=1, keepdims=True)
    sel1 = iota_e == a1
    gate2 = jnp.where(sel1, -1.0, gate)
    m2 = jnp.max(gate2, axis=1, keepdims=True)
    a2 = jnp.min(jnp.where(gate2 == m2, iota_e, E), axis=1, keepdims=True)
    sel2 = iota_e == a2
    wmat_ref[...] = jnp.where(sel1, m1, 0.0) + jnp.where(sel2, m2, 0.0)
    out_ref[...] = jnp.zeros_like(out_ref)

    def start(e):
        slot = lax.rem(e, NBUF)
        pltpu.make_async_copy(W1_hbm.at[e], w1buf.at[slot], sem.at[0, slot]).start()
        pltpu.make_async_copy(W2_hbm.at[e], w2buf.at[slot], sem.at[1, slot]).start()

    for e in range(NBUF):
        start(e)

    def step(e, _):
        slot = lax.rem(e, NBUF)
        pltpu.make_async_copy(W1_hbm.at[e], w1buf.at[slot], sem.at[0, slot]).wait()
        pltpu.make_async_copy(W2_hbm.at[e], w2buf.at[slot], sem.at[1, slot]).wait()

        onehot = (lax.broadcasted_iota(jnp.int32, (E, 1), 0) == e).astype(jnp.float32)
        col = jnp.dot(wmat_ref[...], onehot, preferred_element_type=jnp.float32)
        h = jnp.dot(x_ref[...], w1buf[slot], preferred_element_type=jnp.float32)
        h = jnp.maximum(h + b1_ref[pl.ds(e, 1), :], 0.0)
        y = jnp.dot(h, w2buf[slot], preferred_element_type=jnp.float32)
        y = y + b2_ref[pl.ds(e, 1), :]
        out_ref[...] += col * y

        @pl.when(e + NBUF < E)
        def _next():
            start(e + NBUF)

        return 0

    lax.fori_loop(0, E, step, 0)


def kernel(x, Wg, bg, W1, b1, W2, b2):
    bg2 = bg.reshape(1, E)
    out, gate = pl.pallas_call(
        _moe_body,
        in_specs=[
            pl.BlockSpec((T, HIDDEN), lambda: (0, 0)),
            pl.BlockSpec((HIDDEN, E), lambda: (0, 0)),
            pl.BlockSpec((1, E), lambda: (0, 0)),
            pl.BlockSpec((E, HIDDEN), lambda: (0, 0)),
            pl.BlockSpec((E, HIDDEN), lambda: (0, 0)),
            pl.BlockSpec(memory_space=pl.ANY),
            pl.BlockSpec(memory_space=pl.ANY),
        ],
        out_specs=[
            pl.BlockSpec((T, HIDDEN), lambda: (0, 0)),
            pl.BlockSpec((T, E), lambda: (0, 0)),
        ],
        out_shape=[
            jax.ShapeDtypeStruct((T, HIDDEN), jnp.float32),
            jax.ShapeDtypeStruct((T, E), jnp.float32),
        ],
        scratch_shapes=[
            pltpu.VMEM((NBUF, HIDDEN, HIDDEN), jnp.float32),
            pltpu.VMEM((NBUF, HIDDEN, HIDDEN), jnp.float32),
            pltpu.VMEM((T, E), jnp.float32),
            pltpu.SemaphoreType.DMA((2, NBUF)),
        ],
    )(x, Wg, bg2, b1, b2, W1, W2)
    return (out, gate)
